# baseline (speedup 1.0000x reference)
import jax
import jax.numpy as jnp
from jax.experimental import pallas as pl


_BLK = 2048
_NS = 4


def _mm_body(w_ref, x0, x1, x2, x3, o_ref):
    w = w_ref[...]
    c = 64 // _NS
    acc = jnp.dot(w[:, 0 * c:1 * c], x0[...], preferred_element_type=jnp.float32)
    acc += jnp.dot(w[:, 1 * c:2 * c], x1[...], preferred_element_type=jnp.float32)
    acc += jnp.dot(w[:, 2 * c:3 * c], x2[...], preferred_element_type=jnp.float32)
    acc += jnp.dot(w[:, 3 * c:4 * c], x3[...], preferred_element_type=jnp.float32)
    o_ref[...] = acc


def kernel(x, W):
    B, I = x.shape
    J = W.shape[0]
    xt = x.T
    c = I // _NS
    slices = [jax.lax.slice(xt, (k * c, 0), ((k + 1) * c, B)) for k in range(_NS)]
    xspec = pl.BlockSpec((c, _BLK), lambda g: (0, g))
    outT = pl.pallas_call(
        _mm_body,
        grid=(B // _BLK,),
        in_specs=[pl.BlockSpec((J, I), lambda g: (0, 0))] + [xspec] * _NS,
        out_specs=pl.BlockSpec((J, _BLK), lambda g: (0, g)),
        out_shape=jax.ShapeDtypeStruct((J, B), jnp.float32),
    )(W, *slices)
    return outT.T


# speedup vs baseline: 1.6100x; 1.6100x over previous
import jax
import jax.numpy as jnp
from jax.experimental import pallas as pl


_BLK = 2048
_NS = 4


def _mm_body(w_ref, x0, x1, x2, x3, o_ref):
    w = w_ref[...]
    c = 64 // _NS
    acc = jnp.dot(w[:, 0 * c:1 * c], x0[...], preferred_element_type=jnp.float32)
    acc += jnp.dot(w[:, 1 * c:2 * c], x1[...], preferred_element_type=jnp.float32)
    acc += jnp.dot(w[:, 2 * c:3 * c], x2[...], preferred_element_type=jnp.float32)
    acc += jnp.dot(w[:, 3 * c:4 * c], x3[...], preferred_element_type=jnp.float32)
    o_ref[...] = acc


def kernel(x, W):
    B, I = x.shape
    J = W.shape[0]
    xt = x.T
    c = I // _NS
    xspecs = [pl.BlockSpec((c, _BLK), lambda g, k=k: (k, g)) for k in range(_NS)]
    outT = pl.pallas_call(
        _mm_body,
        grid=(B // _BLK,),
        in_specs=[pl.BlockSpec((J, I), lambda g: (0, 0))] + xspecs,
        out_specs=pl.BlockSpec((J, _BLK), lambda g: (0, g)),
        out_shape=jax.ShapeDtypeStruct((J, B), jnp.float32),
    )(W, *([xt] * _NS))
    return outT.T
